# baseline (speedup 1.0000x reference)
import functools

import jax
import jax.numpy as jnp
from jax import lax
from jax.experimental import pallas as pl
from jax.experimental.pallas import tpu as pltpu
from jax.experimental.pallas import tpu_sc as plsc


def _sc_copy(seq, d, dtype):
    info = plsc.get_sparse_core_info()
    nc, ns = info.num_cores, info.num_subcores
    nw = nc * ns
    rows_per_w = seq // nw
    mesh = plsc.VectorSubcoreMesh(core_axis_name="c", subcore_axis_name="s")

    @functools.partial(
        pl.kernel,
        mesh=mesh,
        out_type=jax.ShapeDtypeStruct((seq, d), dtype),
    )
    def k(emb_hbm, out_hbm):
        wid = lax.axis_index("s") * nc + lax.axis_index("c")
        base = wid * rows_per_w
        pltpu.sync_copy(emb_hbm.at[pl.ds(base, rows_per_w)],
                        out_hbm.at[pl.ds(base, rows_per_w)])

    return k


def kernel(x, emb):
    seq = x.shape[1]
    d = emb.shape[1]
    return _sc_copy(seq, d, emb.dtype)(emb)


# speedup vs baseline: 30.4929x; 30.4929x over previous
import functools

import jax
import jax.numpy as jnp
from jax import lax
from jax.experimental import pallas as pl
from jax.experimental.pallas import tpu as pltpu
from jax.experimental.pallas import tpu_sc as plsc

_CHUNK = 16


def _sc_copy(seq, d, dtype):
    info = plsc.get_sparse_core_info()
    nc, ns = info.num_cores, info.num_subcores
    nw = nc * ns
    rows_per_w = seq // nw
    n_chunks = rows_per_w // _CHUNK
    mesh = plsc.VectorSubcoreMesh(core_axis_name="c", subcore_axis_name="s")

    @functools.partial(
        pl.kernel,
        mesh=mesh,
        out_type=jax.ShapeDtypeStruct((seq, d), dtype),
        scratch_types=[
            pltpu.VMEM((2, _CHUNK, d), dtype),
            pltpu.SemaphoreType.DMA,
            pltpu.SemaphoreType.DMA,
            pltpu.SemaphoreType.DMA,
            pltpu.SemaphoreType.DMA,
        ],
    )
    def k(emb_hbm, out_hbm, buf, si0, si1, so0, so1):
        wid = lax.axis_index("s") * nc + lax.axis_index("c")
        base = wid * rows_per_w
        in_sems = (si0, si1)
        out_sems = (so0, so1)

        def in_copy(i):
            return pltpu.make_async_copy(
                emb_hbm.at[pl.ds(base + i * _CHUNK, _CHUNK)],
                buf.at[i % 2], in_sems[i % 2])

        def out_copy(i):
            return pltpu.make_async_copy(
                buf.at[i % 2],
                out_hbm.at[pl.ds(base + i * _CHUNK, _CHUNK)],
                out_sems[i % 2])

        in_copy(0).start()
        for i in range(n_chunks):
            in_copy(i).wait()
            out_copy(i).start()
            if i + 1 < n_chunks:
                if i >= 1:
                    out_copy(i - 1).wait()
                in_copy(i + 1).start()
        out_copy(n_chunks - 2).wait()
        out_copy(n_chunks - 1).wait()

    return k


def kernel(x, emb):
    seq = x.shape[1]
    d = emb.shape[1]
    return _sc_copy(seq, d, emb.dtype)(emb)


# speedup vs baseline: 43.2997x; 1.4200x over previous
import jax
import jax.numpy as jnp
from jax.experimental import pallas as pl


def _copy_block(emb_ref, o_ref):
    o_ref[...] = emb_ref[...]


def kernel(x, emb):
    seq = x.shape[1]
    d = emb.shape[1]
    block = 256
    return pl.pallas_call(
        _copy_block,
        grid=(seq // block,),
        in_specs=[pl.BlockSpec((block, d), lambda i: (i, 0))],
        out_specs=pl.BlockSpec((block, d), lambda i: (i, 0)),
        out_shape=jax.ShapeDtypeStruct((seq, d), emb.dtype),
    )(emb)


# speedup vs baseline: 48.9846x; 1.1313x over previous
import jax
import jax.numpy as jnp
from jax.experimental import pallas as pl


def _copy_block(emb_ref, o_ref):
    o_ref[...] = emb_ref[...]


def kernel(x, emb):
    seq = x.shape[1]
    d = emb.shape[1]
    block = 1024
    return pl.pallas_call(
        _copy_block,
        grid=(seq // block,),
        in_specs=[pl.BlockSpec((block, d), lambda i: (i, 0))],
        out_specs=pl.BlockSpec((block, d), lambda i: (i, 0)),
        out_shape=jax.ShapeDtypeStruct((seq, d), emb.dtype),
    )(emb)
